# Initial kernel scaffold; baseline (speedup 1.0000x reference)
import jax
import jax.numpy as jnp
from jax.experimental import pallas as pl


def kernel(output, target):
    raise NotImplementedError("write your pallas kernel here")


# speedup vs baseline: 4.4645x; 4.4645x over previous
import math

import jax
import jax.numpy as jnp
from jax.experimental import pallas as pl
from jax.experimental.pallas import tpu as pltpu

_LS = 0.1
_V = 10000
_PAD = 0
_CONF = 1.0 - _LS
_SMOOTH = _LS / (_V - 2)
_C_ROW = (_V - 2) * _SMOOTH * math.log(_SMOOTH) + _CONF * math.log(_CONF)

_BLK = 256


def _loss_body(t_ref, x_ref, o_ref):
    x = x_ref[...]
    rowsum = jnp.sum(x, axis=1, keepdims=True)
    x0 = x[:, 0:1]
    t = t_ref[0]
    cols = jax.lax.broadcasted_iota(jnp.int32, x.shape, 1)
    tval = jnp.sum(jnp.where(cols == t, x, 0.0), axis=1, keepdims=True)
    per_row = _C_ROW - _SMOOTH * (rowsum - x0) - (_CONF - _SMOOTH) * tval
    part = jnp.sum(jnp.where(t != _PAD, per_row, 0.0))

    @pl.when(pl.program_id(0) == 0)
    def _():
        o_ref[0, 0] = 0.0

    o_ref[0, 0] += part


def kernel(output, target):
    n, v = output.shape
    nblk = n // _BLK
    t3 = target.astype(jnp.int32).reshape(nblk, _BLK, 1)
    out = pl.pallas_call(
        _loss_body,
        grid=(nblk,),
        in_specs=[
            pl.BlockSpec((1, _BLK, 1), lambda i: (i, 0, 0)),
            pl.BlockSpec((_BLK, v), lambda i: (i, 0)),
        ],
        out_specs=pl.BlockSpec(memory_space=pltpu.SMEM),
        out_shape=jax.ShapeDtypeStruct((1, 1), jnp.float32),
    )(t3, output)
    return out[0, 0]
